# Initial kernel scaffold; baseline (speedup 1.0000x reference)
import jax
import jax.numpy as jnp
from jax.experimental import pallas as pl


def kernel(x, edges, W1, b1, W2, b2, W3, b3, Wl, bl):
    raise NotImplementedError("write your pallas kernel here")


# speedup vs baseline: 44.7307x; 44.7307x over previous
import functools

import jax
import jax.numpy as jnp
from jax import lax
from jax.experimental import pallas as pl
from jax.experimental.pallas import tpu as pltpu
from jax.experimental.pallas import tpu_sc as plsc

N = 10000
E = 320000
NC = 2
NS = 16
NW = NC * NS
K = 125
EPW = E // NW
CH = EPW // K
N_PAD = 10240
RPT = N_PAD // NS

BLK = 512
GRID = N_PAD // BLK


def _sc_mesh():
    return plsc.VectorSubcoreMesh(core_axis_name="c", subcore_axis_name="s")


_SC_PARAMS = pltpu.CompilerParams(use_tc_tiling_on_sc=False)


def _sc_degree(dst2d, zeros1):
    @functools.partial(
        pl.kernel,
        out_type=jax.ShapeDtypeStruct((NC, N_PAD), jnp.float32),
        mesh=_sc_mesh(),
        compiler_params=_SC_PARAMS,
        scratch_types=[
            pltpu.VMEM((CH, K), jnp.int32),
            pltpu.VMEM((K,), jnp.float32),
            pltpu.VMEM_SHARED((N_PAD,), jnp.float32),
        ],
    )
    def deg_kernel(dst_hbm, zeros_hbm, out_hbm, didx, ones_v, acc):
        c = lax.axis_index("c")
        s = lax.axis_index("s")
        wid = s * NC + c

        def initrow(i, _):
            ones_v[pl.ds(i * 16, 16)] = jnp.full((16,), 1.0, jnp.float32)
            return 0

        lax.fori_loop(0, 7, initrow, 0)
        ones_v[pl.ds(K - 16, 16)] = jnp.full((16,), 1.0, jnp.float32)
        pltpu.sync_copy(zeros_hbm.at[pl.ds(s * RPT, RPT)],
                        acc.at[pl.ds(s * RPT, RPT)])
        pltpu.sync_copy(dst_hbm.at[pl.ds(wid * CH, CH)], didx)
        plsc.subcore_barrier()

        def body(j, _):
            pltpu.sync_copy(ones_v, acc.at[didx.at[j]], add=True)
            return 0

        lax.fori_loop(0, CH, body, 0)
        plsc.subcore_barrier()
        pltpu.sync_copy(acc.at[pl.ds(s * RPT, RPT)],
                        out_hbm.at[c].at[pl.ds(s * RPT, RPT)])

    return deg_kernel(dst2d, zeros1)


def _sc_scatter(g, src2d, dst2d, zeros, F):
    NBUF = 8

    @functools.partial(
        pl.kernel,
        out_type=jax.ShapeDtypeStruct((NC, N_PAD, F), jnp.float32),
        mesh=_sc_mesh(),
        compiler_params=_SC_PARAMS,
        scratch_types=[
            pltpu.VMEM((CH, K), jnp.int32),
            pltpu.VMEM((CH, K), jnp.int32),
            pltpu.VMEM((NBUF, K, F), jnp.float32),
            pltpu.VMEM_SHARED((N_PAD, F), jnp.float32),
            [pltpu.SemaphoreType.DMA] * NBUF,
            [pltpu.SemaphoreType.DMA] * NBUF,
        ],
    )
    def scat_kernel(g_hbm, src_hbm, dst_hbm, zeros_hbm, out_hbm,
                    sidx, didx, rows, acc, gsem, ssem):
        c = lax.axis_index("c")
        s = lax.axis_index("s")
        wid = s * NC + c
        pltpu.sync_copy(zeros_hbm.at[pl.ds(s * RPT, RPT)],
                        acc.at[pl.ds(s * RPT, RPT)])
        pltpu.sync_copy(src_hbm.at[pl.ds(wid * CH, CH)], sidx)
        pltpu.sync_copy(dst_hbm.at[pl.ds(wid * CH, CH)], didx)
        plsc.subcore_barrier()

        def gather(j, b):
            pltpu.async_copy(g_hbm.at[sidx.at[j]], rows.at[b], gsem[b])

        def gather_wait(b):
            pltpu.make_async_copy(g_hbm.at[sidx.at[0]], rows.at[b],
                                  gsem[b]).wait()

        def scat(j, b):
            pltpu.async_copy(rows.at[b], acc.at[didx.at[j]], ssem[b],
                             add=True)

        def scat_wait(b):
            pltpu.make_async_copy(rows.at[b], acc.at[didx.at[0]],
                                  ssem[b]).wait()

        for b in range(NBUF):
            gather(b, b)

        def outer(t, _):
            base = t * NBUF
            for b in range(NBUF):
                gather_wait(b)
                scat(base + b, b)
            for b in range(NBUF):
                nxt = base + NBUF + b

                @pl.when(nxt < CH)
                def _():
                    scat_wait(b)
                    gather(nxt, b)
            return 0

        lax.fori_loop(0, CH // NBUF, outer, 0)
        for b in range(NBUF):
            scat_wait(b)
        plsc.subcore_barrier()
        pltpu.sync_copy(acc.at[pl.ds(s * RPT, RPT)],
                        out_hbm.at[c].at[pl.ds(s * RPT, RPT)])

    return scat_kernel(g, src2d, dst2d, zeros)


def _dis_block(degr_ref):
    deg = 1.0 + degr_ref[0, 0, 0] + degr_ref[1, 0, 0]
    return lax.rsqrt(deg)[:, None]


def _tc_first(degR, x, W1):
    def body(degr_ref, x_ref, w_ref, out_ref):
        dis = _dis_block(degr_ref)
        out_ref[...] = dis * jnp.dot(x_ref[...], w_ref[...],
                                     preferred_element_type=jnp.float32)

    return pl.pallas_call(
        body,
        grid=(GRID,),
        in_specs=[
            pl.BlockSpec((NC, 1, 1, BLK), lambda i: (0, i, 0, 0)),
            pl.BlockSpec((BLK, 128), lambda i: (i, 0)),
            pl.BlockSpec((128, 64), lambda i: (0, 0)),
        ],
        out_specs=pl.BlockSpec((BLK, 64), lambda i: (i, 0)),
        out_shape=jax.ShapeDtypeStruct((N, 64), jnp.float32),
    )(degR, x, W1)


def _tc_mid(degR, P, g, b, W, F, F2):
    def body(degr_ref, p_ref, g_ref, b_ref, w_ref, out_ref):
        dis = _dis_block(degr_ref)
        xn = jnp.maximum(dis * (p_ref[0] + p_ref[1] + g_ref[...]) + b_ref[...],
                         0.0)
        out_ref[...] = dis * jnp.dot(xn, w_ref[...],
                                     preferred_element_type=jnp.float32)

    return pl.pallas_call(
        body,
        grid=(GRID,),
        in_specs=[
            pl.BlockSpec((NC, 1, 1, BLK), lambda i: (0, i, 0, 0)),
            pl.BlockSpec((NC, BLK, F), lambda i: (0, i, 0)),
            pl.BlockSpec((BLK, F), lambda i: (i, 0)),
            pl.BlockSpec((1, F), lambda i: (0, 0)),
            pl.BlockSpec((F, F2), lambda i: (0, 0)),
        ],
        out_specs=pl.BlockSpec((BLK, F2), lambda i: (i, 0)),
        out_shape=jax.ShapeDtypeStruct((N, F2), jnp.float32),
    )(degR, P, g, b, W)


def _tc_final(degR, P, g3, b3, Wl, bl):
    def body(degr_ref, p_ref, g_ref, b_ref, wl_ref, bl_ref, h_ref, z_ref):
        dis = _dis_block(degr_ref)
        h = jnp.maximum(dis * (p_ref[0] + p_ref[1] + g_ref[...]) + b_ref[...],
                        0.0)
        h_ref[...] = h
        z_ref[...] = jnp.dot(h, wl_ref[...],
                             preferred_element_type=jnp.float32) + bl_ref[...]

    return pl.pallas_call(
        body,
        grid=(GRID,),
        in_specs=[
            pl.BlockSpec((NC, 1, 1, BLK), lambda i: (0, i, 0, 0)),
            pl.BlockSpec((NC, BLK, 16), lambda i: (0, i, 0)),
            pl.BlockSpec((BLK, 16), lambda i: (i, 0)),
            pl.BlockSpec((1, 16), lambda i: (0, 0)),
            pl.BlockSpec((16, 4), lambda i: (0, 0)),
            pl.BlockSpec((1, 4), lambda i: (0, 0)),
        ],
        out_specs=[
            pl.BlockSpec((BLK, 16), lambda i: (i, 0)),
            pl.BlockSpec((BLK, 4), lambda i: (i, 0)),
        ],
        out_shape=[
            jax.ShapeDtypeStruct((N, 16), jnp.float32),
            jax.ShapeDtypeStruct((N, 4), jnp.float32),
        ],
    )(degR, P, g3, b3, Wl, bl)


def kernel(x, edges, W1, b1, W2, b2, W3, b3, Wl, bl):
    src2d = edges[0].reshape(E // K, K)
    dst2d = edges[1].reshape(E // K, K)
    zeros1 = jnp.zeros((N_PAD,), jnp.float32)
    zeros16 = jnp.zeros((N_PAD, 16), jnp.float32)
    zeros32 = jnp.zeros((N_PAD, 32), jnp.float32)
    zeros64 = jnp.zeros((N_PAD, 64), jnp.float32)

    degP = _sc_degree(dst2d, zeros1)
    degR = degP.reshape(NC, GRID, 1, BLK)

    g1 = _tc_first(degR, x, W1)
    P1 = _sc_scatter(g1, src2d, dst2d, zeros64, 64)
    g2 = _tc_mid(degR, P1, g1, b1.reshape(1, 64), W2, 64, 32)
    P2 = _sc_scatter(g2, src2d, dst2d, zeros32, 32)
    g3 = _tc_mid(degR, P2, g2, b2.reshape(1, 32), W3, 32, 16)
    P3 = _sc_scatter(g3, src2d, dst2d, zeros16, 16)
    h, z = _tc_final(degR, P3, g3, b3.reshape(1, 16), Wl, bl.reshape(1, 4))
    return (h, z)
